# Initial kernel scaffold; baseline (speedup 1.0000x reference)
import jax
import jax.numpy as jnp
from jax.experimental import pallas as pl


def kernel(x, y):
    raise NotImplementedError("write your pallas kernel here")


# speedup vs baseline: 36.5961x; 36.5961x over previous
import functools

import jax
import jax.numpy as jnp
from jax import lax
from jax.experimental import pallas as pl
from jax.experimental.pallas import tpu as pltpu
from jax.experimental.pallas import tpu_sc as plsc

_NUM_BINS = 256
_B = 16
_PIX = 3 * 512 * 512
_HALF = _PIX // 2
_CHUNK = 32768
_NCHUNK = _HALF // _CHUNK
_LANES = 16
_VECS = _CHUNK // _LANES
_NW = 32


def _hist_body(x_hbm, y_hbm, out_hbm, buf0, buf1, hist, sem0, sem1):
    wid = lax.axis_index("s") * 2 + lax.axis_index("c")
    img = wid % _B
    half = wid // _B
    base = img * _PIX + half * _HALF

    ones = jnp.full((_LANES,), 1.0, jnp.float32)
    zeros = jnp.zeros((_LANES,), jnp.float32)

    def zero_body(i, _):
        hist[pl.ds(i * _LANES, _LANES)] = zeros
        return 0

    lax.fori_loop(0, (2 * _NUM_BINS) // _LANES, zero_body, 0)

    bufs = (buf0, buf1)
    sems = (sem0, sem1)
    srcs = [(x_hbm, 0)] * _NCHUNK + [(y_hbm, _NUM_BINS)] * _NCHUNK

    def start(i):
        src, _ = srcs[i]
        off = base + (i % _NCHUNK) * _CHUNK
        return pltpu.async_copy(src.at[pl.ds(off, _CHUNK)], bufs[i % 2],
                                sems[i % 2])

    def consume(buf, bin_off):
        def body(i, _):
            v = buf[pl.ds(i * _LANES, _LANES)]
            t = ((v * 255.0) / 255.0) * 256.0
            idx = t.astype(jnp.int32)
            idx = jnp.minimum(jnp.maximum(idx, 0), _NUM_BINS - 1) + bin_off
            plsc.addupdate_scatter(hist, [idx], ones)
            return 0

        lax.fori_loop(0, _VECS, body, 0)

    cp = start(0)
    for i in range(2 * _NCHUNK):
        nxt = start(i + 1) if i + 1 < 2 * _NCHUNK else None
        cp.wait()
        consume(bufs[i % 2], srcs[i][1])
        cp = nxt

    pltpu.sync_copy(hist, out_hbm.at[wid])


_hist_kernel = functools.partial(
    pl.kernel,
    out_type=jax.ShapeDtypeStruct((_NW, 2 * _NUM_BINS), jnp.float32),
    mesh=plsc.VectorSubcoreMesh(core_axis_name="c", subcore_axis_name="s"),
    compiler_params=pltpu.CompilerParams(needs_layout_passes=False),
    scratch_types=[
        pltpu.VMEM((_CHUNK,), jnp.float32),
        pltpu.VMEM((_CHUNK,), jnp.float32),
        pltpu.VMEM((2 * _NUM_BINS,), jnp.float32),
        pltpu.SemaphoreType.DMA,
        pltpu.SemaphoreType.DMA,
    ],
)(_hist_body)


def _loss_body(h_ref, o_ref):
    h = h_ref[...]
    full = h[0:_B, :] + h[_B:2 * _B, :]
    hn = full / jnp.float32(_PIX)
    s = jnp.sqrt(hn)
    col = lax.broadcasted_iota(jnp.int32, (_B, 2 * _NUM_BINS), 1)
    sgn = jnp.where(col < _NUM_BINS, jnp.float32(1.0), jnp.float32(-1.0))
    d = jnp.sum(s * sgn)
    o_ref[...] = jnp.reshape(jnp.clip(d * d, 0.0, 1.0), (1, 1))


def kernel(x, y):
    hists = _hist_kernel(x.reshape(-1), y.reshape(-1))
    loss = pl.pallas_call(
        _loss_body,
        out_shape=jax.ShapeDtypeStruct((1, 1), jnp.float32),
    )(hists)
    return jnp.reshape(loss, ())


# speedup vs baseline: 106.9771x; 2.9232x over previous
import functools

import jax
import jax.numpy as jnp
from jax import lax
from jax.experimental import pallas as pl
from jax.experimental.pallas import tpu as pltpu
from jax.experimental.pallas import tpu_sc as plsc

_NUM_BINS = 256
_B = 16
_PIX = 3 * 512 * 512
_HALF = _PIX // 2
_CHUNK = 32768
_NCHUNK = _HALF // _CHUNK
_LANES = 16
_VECS = _CHUNK // _LANES
_NW = 32
_U = 8
_SUB = 2 * _NUM_BINS
_HBINS = _U * _SUB


def _hist_body(x_hbm, y_hbm, out_hbm, buf0, buf1, hist, sem0, sem1):
    wid = lax.axis_index("s") * 2 + lax.axis_index("c")
    img = wid % _B
    half = wid // _B
    base = img * _PIX + half * _HALF

    ones = jnp.full((_LANES,), 1.0, jnp.float32)
    zeros = jnp.zeros((_LANES,), jnp.float32)

    def zero_body(i, _):
        hist[pl.ds(i * _LANES, _LANES)] = zeros
        return 0

    lax.fori_loop(0, _HBINS // _LANES, zero_body, 0)

    bufs = (buf0, buf1)
    sems = (sem0, sem1)
    srcs = [(x_hbm, 0)] * _NCHUNK + [(y_hbm, _NUM_BINS)] * _NCHUNK

    def start(i):
        src, _ = srcs[i]
        off = base + (i % _NCHUNK) * _CHUNK
        return pltpu.async_copy(src.at[pl.ds(off, _CHUNK)], bufs[i % 2],
                                sems[i % 2])

    def consume(buf, bin_off):
        def body(i, _):
            b0 = i * (_LANES * _U)
            vs = [buf[pl.ds(b0 + j * _LANES, _LANES)] for j in range(_U)]
            idxs = []
            for j, v in enumerate(vs):
                t = ((v * 255.0) / 255.0) * 256.0
                idx = t.astype(jnp.int32)
                idx = jnp.minimum(idx, _NUM_BINS - 1)
                idxs.append(idx + (j * _SUB + bin_off))
            for idx in idxs:
                plsc.addupdate_scatter(hist, [idx], ones)
            return 0

        lax.fori_loop(0, _VECS // _U, body, 0)

    cp = start(0)
    for i in range(2 * _NCHUNK):
        nxt = start(i + 1) if i + 1 < 2 * _NCHUNK else None
        cp.wait()
        consume(bufs[i % 2], srcs[i][1])
        cp = nxt

    pltpu.sync_copy(hist, out_hbm.at[wid])


_hist_kernel = functools.partial(
    pl.kernel,
    out_type=jax.ShapeDtypeStruct((_NW, _HBINS), jnp.float32),
    mesh=plsc.VectorSubcoreMesh(core_axis_name="c", subcore_axis_name="s"),
    compiler_params=pltpu.CompilerParams(needs_layout_passes=False),
    scratch_types=[
        pltpu.VMEM((_CHUNK,), jnp.float32),
        pltpu.VMEM((_CHUNK,), jnp.float32),
        pltpu.VMEM((_HBINS,), jnp.float32),
        pltpu.SemaphoreType.DMA,
        pltpu.SemaphoreType.DMA,
    ],
)(_hist_body)


def _loss_body(h_ref, o_ref):
    h = h_ref[...]
    acc = h[:, 0:_SUB]
    for j in range(1, _U):
        acc = acc + h[:, j * _SUB:(j + 1) * _SUB]
    full = acc[0:_B, :] + acc[_B:2 * _B, :]
    hn = full / jnp.float32(_PIX)
    s = jnp.sqrt(hn)
    col = lax.broadcasted_iota(jnp.int32, (_B, _SUB), 1)
    sgn = jnp.where(col < _NUM_BINS, jnp.float32(1.0), jnp.float32(-1.0))
    d = jnp.sum(s * sgn)
    o_ref[...] = jnp.reshape(jnp.clip(d * d, 0.0, 1.0), (1, 1))


def kernel(x, y):
    hists = _hist_kernel(x.reshape(-1), y.reshape(-1))
    loss = pl.pallas_call(
        _loss_body,
        out_shape=jax.ShapeDtypeStruct((1, 1), jnp.float32),
    )(hists)
    return jnp.reshape(loss, ())


# speedup vs baseline: 141.6289x; 1.3239x over previous
import functools

import jax
import jax.numpy as jnp
from jax import lax
from jax.experimental import pallas as pl
from jax.experimental.pallas import tpu as pltpu
from jax.experimental.pallas import tpu_sc as plsc

_NUM_BINS = 256
_B = 16
_W = 512
_ROWS = 16 * 3 * 512
_RPI = _ROWS // _B
_RPW = _RPI // 2
_CROWS = 64
_NCHUNK = _RPW // _CROWS
_LANES = 16
_U = 8
_SUB = 2 * _NUM_BINS
_HBINS = _U * _SUB
_NW = 32
_PIX = _RPI * _W


def _hist_body(x_hbm, y_hbm, out_hbm, buf0, buf1, hist, sem0, sem1):
    wid = lax.axis_index("s") * 2 + lax.axis_index("c")
    row0 = wid * _RPW
    out_row = (wid % 2) * _B + wid // 2

    ones = jnp.full((_LANES,), 1.0, jnp.float32)
    zeros = jnp.zeros((_LANES,), jnp.float32)

    def zero_body(i, _):
        hist[pl.ds(i * _LANES, _LANES)] = zeros
        return 0

    lax.fori_loop(0, _HBINS // _LANES, zero_body, 0)

    bufs = (buf0, buf1)
    sems = (sem0, sem1)
    srcs = [(x_hbm, 0)] * _NCHUNK + [(y_hbm, _NUM_BINS)] * _NCHUNK

    def start(i):
        src, _ = srcs[i]
        r = row0 + (i % _NCHUNK) * _CROWS
        return pltpu.async_copy(src.at[pl.ds(r, _CROWS), :], bufs[i % 2],
                                sems[i % 2])

    def consume(buf, bin_off):
        def body(i, _):
            r = i >> 2
            cb = (i & 3) * (_U * _LANES)
            vs = [buf[r, pl.ds(cb + j * _LANES, _LANES)] for j in range(_U)]
            idxs = []
            for j, v in enumerate(vs):
                t = ((v * 255.0) / 255.0) * 256.0
                idx = t.astype(jnp.int32)
                idx = jnp.minimum(idx, _NUM_BINS - 1)
                idxs.append(idx + (j * _SUB + bin_off))
            for idx in idxs:
                plsc.addupdate_scatter(hist, [idx], ones)
            return 0

        lax.fori_loop(0, (_CROWS * _W) // (_LANES * _U), body, 0)

    cp = start(0)
    for i in range(2 * _NCHUNK):
        nxt = start(i + 1) if i + 1 < 2 * _NCHUNK else None
        cp.wait()
        consume(bufs[i % 2], srcs[i][1])
        cp = nxt

    pltpu.sync_copy(hist, out_hbm.at[out_row])


_hist_kernel = functools.partial(
    pl.kernel,
    out_type=jax.ShapeDtypeStruct((_NW, _HBINS), jnp.float32),
    mesh=plsc.VectorSubcoreMesh(core_axis_name="c", subcore_axis_name="s"),
    compiler_params=pltpu.CompilerParams(needs_layout_passes=False),
    scratch_types=[
        pltpu.VMEM((_CROWS, _W), jnp.float32),
        pltpu.VMEM((_CROWS, _W), jnp.float32),
        pltpu.VMEM((_HBINS,), jnp.float32),
        pltpu.SemaphoreType.DMA,
        pltpu.SemaphoreType.DMA,
    ],
)(_hist_body)


def _loss_body(h_ref, o_ref):
    h = h_ref[...]
    acc = h[:, 0:_SUB]
    for j in range(1, _U):
        acc = acc + h[:, j * _SUB:(j + 1) * _SUB]
    full = acc[0:_B, :] + acc[_B:2 * _B, :]
    hn = full / jnp.float32(_PIX)
    s = jnp.sqrt(hn)
    col = lax.broadcasted_iota(jnp.int32, (_B, _SUB), 1)
    sgn = jnp.where(col < _NUM_BINS, jnp.float32(1.0), jnp.float32(-1.0))
    d = jnp.sum(s * sgn)
    o_ref[...] = jnp.reshape(jnp.clip(d * d, 0.0, 1.0), (1, 1))


def kernel(x, y):
    hists = _hist_kernel(x.reshape(_ROWS, _W), y.reshape(_ROWS, _W))
    loss = pl.pallas_call(
        _loss_body,
        out_shape=jax.ShapeDtypeStruct((1, 1), jnp.float32),
    )(hists)
    return jnp.reshape(loss, ())


# speedup vs baseline: 149.6499x; 1.0566x over previous
import functools

import jax
import jax.numpy as jnp
from jax import lax
from jax.experimental import pallas as pl
from jax.experimental.pallas import tpu as pltpu
from jax.experimental.pallas import tpu_sc as plsc

_NUM_BINS = 256
_B = 16
_W = 512
_ROWS = 16 * 3 * 512
_RPI = _ROWS // _B
_RPW = _RPI // 2
_CROWS = 64
_NCHUNK = _RPW // _CROWS
_LANES = 16
_U = 8
_SUB = 2 * _NUM_BINS
_HBINS = _U * _SUB
_NW = 32
_PIX = _RPI * _W


def _hist_body(x_hbm, y_hbm, out_hbm, buf0, buf1, hist, sem0, sem1):
    wid = lax.axis_index("s") * 2 + lax.axis_index("c")
    row0 = wid * _RPW
    out_row = (wid % 2) * _B + wid // 2

    ones = jnp.full((_LANES,), 1.0, jnp.float32)
    zeros = jnp.zeros((_LANES,), jnp.float32)

    def zero_body(i, _):
        hist[pl.ds(i * _LANES, _LANES)] = zeros
        return 0

    lax.fori_loop(0, _HBINS // _LANES, zero_body, 0)

    bufs = (buf0, buf1)
    sems = (sem0, sem1)
    nch = 2 * _NCHUNK

    def start(chunk, b):
        r = row0 + (chunk - _NCHUNK * (chunk >= _NCHUNK)) * _CROWS

        @pl.when(chunk < _NCHUNK)
        def _():
            pltpu.async_copy(x_hbm.at[pl.ds(r, _CROWS), :], bufs[b], sems[b])

        @pl.when(jnp.logical_and(chunk >= _NCHUNK, chunk < nch))
        def _():
            pltpu.async_copy(y_hbm.at[pl.ds(r, _CROWS), :], bufs[b],
                             sems[b])

    def consume(buf, bin_off):
        def body(i, _):
            r = i >> 2
            cb = (i & 3) * (_U * _LANES)
            vs = [buf[r, pl.ds(cb + j * _LANES, _LANES)] for j in range(_U)]
            idxs = []
            for j, v in enumerate(vs):
                t = ((v * 255.0) / 255.0) * 256.0
                t = jnp.minimum(t, jnp.float32(_NUM_BINS - 1))
                idx = t.astype(jnp.int32)
                idxs.append(idx + (j * _SUB + bin_off))
            for idx in idxs:
                plsc.addupdate_scatter(hist, [idx], ones)
            return 0

        lax.fori_loop(0, (_CROWS * _W) // (_LANES * _U), body, 0)

    start(jnp.int32(0), 0)
    start(jnp.int32(1), 1)

    def chunk_body(c2, _):
        chunk = c2 * 2
        for b in range(2):
            ck = chunk + b
            pltpu.make_async_copy(x_hbm.at[pl.ds(0, _CROWS), :], bufs[b],
                                  sems[b]).wait()
            bin_off = jnp.where(ck < _NCHUNK, 0, _NUM_BINS)
            consume(bufs[b], bin_off)
            start(ck + 2, b)
        return 0

    lax.fori_loop(0, _NCHUNK, chunk_body, 0)

    pltpu.sync_copy(hist, out_hbm.at[out_row])


_hist_kernel = functools.partial(
    pl.kernel,
    out_type=jax.ShapeDtypeStruct((_NW, _HBINS), jnp.float32),
    mesh=plsc.VectorSubcoreMesh(core_axis_name="c", subcore_axis_name="s"),
    compiler_params=pltpu.CompilerParams(needs_layout_passes=False),
    scratch_types=[
        pltpu.VMEM((_CROWS, _W), jnp.float32),
        pltpu.VMEM((_CROWS, _W), jnp.float32),
        pltpu.VMEM((_HBINS,), jnp.float32),
        pltpu.SemaphoreType.DMA,
        pltpu.SemaphoreType.DMA,
    ],
)(_hist_body)


def _loss_body(h_ref, o_ref):
    h = h_ref[...]
    acc = h[:, 0:_SUB]
    for j in range(1, _U):
        acc = acc + h[:, j * _SUB:(j + 1) * _SUB]
    full = acc[0:_B, :] + acc[_B:2 * _B, :]
    hn = full / jnp.float32(_PIX)
    s = jnp.sqrt(hn)
    d = jnp.sum(s[:, 0:_NUM_BINS] - s[:, _NUM_BINS:_SUB])
    o_ref[...] = jnp.reshape(jnp.clip(d * d, 0.0, 1.0), (1, 1))


def kernel(x, y):
    hists = _hist_kernel(x.reshape(_ROWS, _W), y.reshape(_ROWS, _W))
    loss = pl.pallas_call(
        _loss_body,
        out_shape=jax.ShapeDtypeStruct((1, 1), jnp.float32),
    )(hists)
    return jnp.reshape(loss, ())


# speedup vs baseline: 207.7587x; 1.3883x over previous
import functools

import jax
import jax.numpy as jnp
from jax import lax
from jax.experimental import pallas as pl
from jax.experimental.pallas import tpu as pltpu
from jax.experimental.pallas import tpu_sc as plsc

_NUM_BINS = 256
_B = 16
_W = 512
_ROWS = 16 * 3 * 512
_RPI = _ROWS // _B
_RPW = _RPI // 2
_CROWS = 64
_NCHUNK = _RPW // _CROWS
_LANES = 16
_U = 8
_SUB = 2 * _NUM_BINS
_HBINS = _U * _SUB
_NW = 32
_PIX = _RPI * _W


def _hist_body(x_hbm, y_hbm, out_hbm, buf0, buf1, hist, sem0, sem1):
    wid = lax.axis_index("s") * 2 + lax.axis_index("c")
    row0 = wid * _RPW
    out_row = (wid % 2) * _B + wid // 2

    ones = jnp.full((_LANES,), 1.0, jnp.float32)
    zeros = jnp.zeros((_LANES,), jnp.float32)

    def zero_body(i, _):
        hist[pl.ds(i * _LANES, _LANES)] = zeros
        return 0

    lax.fori_loop(0, _HBINS // _LANES, zero_body, 0)

    bufs = (buf0, buf1)
    sems = (sem0, sem1)
    nch = 2 * _NCHUNK

    def start(chunk, b):
        r = row0 + (chunk - _NCHUNK * (chunk >= _NCHUNK)) * _CROWS

        @pl.when(chunk < _NCHUNK)
        def _():
            pltpu.async_copy(x_hbm.at[pl.ds(r, _CROWS), :], bufs[b], sems[b])

        @pl.when(jnp.logical_and(chunk >= _NCHUNK, chunk < nch))
        def _():
            pltpu.async_copy(y_hbm.at[pl.ds(r, _CROWS), :], bufs[b],
                             sems[b])

    def consume(buf, bin_off):
        nbody = (_CROWS * _W) // (_LANES * _U)

        def load8(i):
            r = i >> 2
            cb = (i & 3) * (_U * _LANES)
            return tuple(buf[r, pl.ds(cb + j * _LANES, _LANES)]
                         for j in range(_U))

        def do8(vs):
            idxs = []
            for v in vs:
                t = ((v * 255.0) / 255.0) * 256.0
                t = jnp.minimum(t, jnp.float32(_NUM_BINS - 1))
                idxs.append(t.astype(jnp.int32))
            for j, idx in enumerate(idxs):
                plsc.addupdate_scatter(
                    hist.at[pl.ds(bin_off + j * _SUB, _NUM_BINS)], [idx],
                    ones)

        def body(i, vs):
            nxt = load8(i + 1)
            do8(vs)
            return nxt

        do8(lax.fori_loop(0, nbody - 1, body, load8(0)))

    start(jnp.int32(0), 0)
    start(jnp.int32(1), 1)

    def chunk_body(c2, _):
        chunk = c2 * 2
        for b in range(2):
            ck = chunk + b
            pltpu.make_async_copy(x_hbm.at[pl.ds(0, _CROWS), :], bufs[b],
                                  sems[b]).wait()
            bin_off = jnp.where(ck < _NCHUNK, 0, _NUM_BINS)
            consume(bufs[b], bin_off)
            start(ck + 2, b)
        return 0

    lax.fori_loop(0, _NCHUNK, chunk_body, 0)

    pltpu.sync_copy(hist, out_hbm.at[out_row])


_hist_kernel = functools.partial(
    pl.kernel,
    out_type=jax.ShapeDtypeStruct((_NW, _HBINS), jnp.float32),
    mesh=plsc.VectorSubcoreMesh(core_axis_name="c", subcore_axis_name="s"),
    compiler_params=pltpu.CompilerParams(needs_layout_passes=False),
    scratch_types=[
        pltpu.VMEM((_CROWS, _W), jnp.float32),
        pltpu.VMEM((_CROWS, _W), jnp.float32),
        pltpu.VMEM((_HBINS,), jnp.float32),
        pltpu.SemaphoreType.DMA,
        pltpu.SemaphoreType.DMA,
    ],
)(_hist_body)


def _loss_body(h_ref, o_ref):
    h = h_ref[...]
    acc = h[:, 0:_SUB]
    for j in range(1, _U):
        acc = acc + h[:, j * _SUB:(j + 1) * _SUB]
    full = acc[0:_B, :] + acc[_B:2 * _B, :]
    hn = full / jnp.float32(_PIX)
    s = jnp.sqrt(hn)
    d = jnp.sum(s[:, 0:_NUM_BINS] - s[:, _NUM_BINS:_SUB])
    o_ref[...] = jnp.reshape(jnp.clip(d * d, 0.0, 1.0), (1, 1))


def kernel(x, y):
    hists = _hist_kernel(x.reshape(_ROWS, _W), y.reshape(_ROWS, _W))
    loss = pl.pallas_call(
        _loss_body,
        out_shape=jax.ShapeDtypeStruct((1, 1), jnp.float32),
    )(hists)
    return jnp.reshape(loss, ())
